# baseline (speedup 1.0000x reference)
import jax
import jax.numpy as jnp
from jax.experimental import pallas as pl

_K = 16
_D = 1024
_d = 64
_BT = 512

_PREC = jax.lax.Precision.HIGHEST


def _prep_kernel(us_ref, v_ref):
    a_list = []
    for k in range(_K):
        u = us_ref[k]
        a_list.append(
            jax.lax.dot_general(u, u, (((0,), (0,)), ((), ())),
                                preferred_element_type=jnp.float32,
                                precision=_PREC))
    a = jnp.stack(a_list, axis=0)

    rows = jax.lax.broadcasted_iota(jnp.int32, (_K, _d, _d), 1)
    cols = jax.lax.broadcasted_iota(jnp.int32, (_K, _d, _d), 2)
    eye = (rows == cols).astype(jnp.float32)

    def gj_step(j, carry):
        m, inv = carry
        mask_r = (rows == j).astype(jnp.float32)
        mask_c = (cols == j).astype(jnp.float32)
        row_m = jnp.sum(m * mask_r, axis=1, keepdims=True)
        row_i = jnp.sum(inv * mask_r, axis=1, keepdims=True)
        piv = jnp.sum(row_m * mask_c[:, :1, :], axis=2, keepdims=True)
        inv_piv = 1.0 / piv
        row_mn = row_m * inv_piv
        row_in = row_i * inv_piv
        col_m = jnp.sum(m * mask_c, axis=2, keepdims=True)
        m_new = m - col_m * row_mn + mask_r * row_mn
        inv_new = inv - col_m * row_in + mask_r * row_in
        return m_new, inv_new

    _, a_inv = jax.lax.fori_loop(0, _d, gj_step, (a, eye))

    for k in range(_K):
        u = us_ref[k]
        v_ref[k] = jax.lax.dot_general(u, a_inv[k], (((1,), (0,)), ((), ())),
                                       preferred_element_type=jnp.float32,
                                       precision=_PREC)


def _apply_kernel(x_ref, us_ref, v_ref, o_ref):
    t = jax.lax.dot_general(x_ref[...], us_ref[0], (((1,), (0,)), ((), ())),
                            preferred_element_type=jnp.float32,
                            precision=_PREC)
    o_ref[0] = jax.lax.dot_general(t, v_ref[0], (((1,), (1,)), ((), ())),
                                   preferred_element_type=jnp.float32,
                                   precision=_PREC)


def kernel(x, Us):
    batch = x.shape[0]
    n_bt = batch // _BT

    v = pl.pallas_call(
        _prep_kernel,
        out_shape=jax.ShapeDtypeStruct((_K, _D, _d), jnp.float32),
    )(Us)

    out = pl.pallas_call(
        _apply_kernel,
        grid=(n_bt, _K),
        in_specs=[
            pl.BlockSpec((_BT, _D), lambda i, k: (i, 0)),
            pl.BlockSpec((1, _D, _d), lambda i, k: (k, 0, 0)),
            pl.BlockSpec((1, _D, _d), lambda i, k: (k, 0, 0)),
        ],
        out_specs=pl.BlockSpec((1, _BT, _D), lambda i, k: (k, i, 0)),
        out_shape=jax.ShapeDtypeStruct((_K, batch, _D), jnp.float32),
    )(x, Us, v)
    return out


# speedup vs baseline: 4.2218x; 4.2218x over previous
import jax
import jax.numpy as jnp
from jax.experimental import pallas as pl

_K = 16
_D = 1024
_d = 64
_BT = 1024

_PREC = jax.lax.Precision.HIGHEST
_PREC_APPLY = jax.lax.Precision.DEFAULT


def _prep_kernel(us_ref, v_ref):
    a_list = []
    for k in range(_K):
        u = us_ref[k]
        a_list.append(
            jax.lax.dot_general(u, u, (((0,), (0,)), ((), ())),
                                preferred_element_type=jnp.float32,
                                precision=_PREC))
    a = jnp.stack(a_list, axis=0)

    rows = jax.lax.broadcasted_iota(jnp.int32, (_K, _d, _d), 1)
    cols = jax.lax.broadcasted_iota(jnp.int32, (_K, _d, _d), 2)
    eye = (rows == cols).astype(jnp.float32)

    def gj_step(j, carry):
        m, inv = carry
        mask_r = (rows == j).astype(jnp.float32)
        mask_c = (cols == j).astype(jnp.float32)
        row_m = jnp.sum(m * mask_r, axis=1, keepdims=True)
        row_i = jnp.sum(inv * mask_r, axis=1, keepdims=True)
        piv = jnp.sum(row_m * mask_c[:, :1, :], axis=2, keepdims=True)
        inv_piv = 1.0 / piv
        row_mn = row_m * inv_piv
        row_in = row_i * inv_piv
        col_m = jnp.sum(m * mask_c, axis=2, keepdims=True)
        m_new = m - col_m * row_mn + mask_r * row_mn
        inv_new = inv - col_m * row_in + mask_r * row_in
        return m_new, inv_new

    _, a_inv = jax.lax.fori_loop(0, _d, gj_step, (a, eye))

    for k in range(_K):
        u = us_ref[k]
        v_ref[k] = jax.lax.dot_general(u, a_inv[k], (((1,), (0,)), ((), ())),
                                       preferred_element_type=jnp.float32,
                                       precision=_PREC)


def _apply_kernel(x_ref, us_ref, v_ref, o_ref):
    t = jax.lax.dot_general(x_ref[...], us_ref[0], (((1,), (0,)), ((), ())),
                            preferred_element_type=jnp.float32,
                            precision=_PREC_APPLY)
    o_ref[0] = jax.lax.dot_general(t, v_ref[0], (((1,), (1,)), ((), ())),
                                   preferred_element_type=jnp.float32,
                                   precision=_PREC_APPLY)


def kernel(x, Us):
    batch = x.shape[0]
    n_bt = batch // _BT

    v = pl.pallas_call(
        _prep_kernel,
        out_shape=jax.ShapeDtypeStruct((_K, _D, _d), jnp.float32),
    )(Us)

    out = pl.pallas_call(
        _apply_kernel,
        grid=(n_bt, _K),
        in_specs=[
            pl.BlockSpec((_BT, _D), lambda i, k: (i, 0)),
            pl.BlockSpec((1, _D, _d), lambda i, k: (k, 0, 0)),
            pl.BlockSpec((1, _D, _d), lambda i, k: (k, 0, 0)),
        ],
        out_specs=pl.BlockSpec((1, _BT, _D), lambda i, k: (k, i, 0)),
        out_shape=jax.ShapeDtypeStruct((_K, batch, _D), jnp.float32),
    )(x, Us, v)
    return out


# speedup vs baseline: 4.5515x; 1.0781x over previous
import jax
import jax.numpy as jnp
from jax.experimental import pallas as pl

_K = 16
_D = 1024
_d = 64
_BT = 2048

_PREC = jax.lax.Precision.HIGHEST
_PREC_APPLY = jax.lax.Precision.DEFAULT


def _prep_kernel(us_ref, v_ref):
    a_list = []
    for k in range(_K):
        u = us_ref[k]
        a_list.append(
            jax.lax.dot_general(u, u, (((0,), (0,)), ((), ())),
                                preferred_element_type=jnp.float32,
                                precision=_PREC))
    a = jnp.stack(a_list, axis=0)

    rows = jax.lax.broadcasted_iota(jnp.int32, (_K, _d, _d), 1)
    cols = jax.lax.broadcasted_iota(jnp.int32, (_K, _d, _d), 2)
    eye = (rows == cols).astype(jnp.float32)

    def gj_step(j, carry):
        m, inv = carry
        mask_r = (rows == j).astype(jnp.float32)
        mask_c = (cols == j).astype(jnp.float32)
        row_m = jnp.sum(m * mask_r, axis=1, keepdims=True)
        row_i = jnp.sum(inv * mask_r, axis=1, keepdims=True)
        piv = jnp.sum(row_m * mask_c[:, :1, :], axis=2, keepdims=True)
        inv_piv = 1.0 / piv
        row_mn = row_m * inv_piv
        row_in = row_i * inv_piv
        col_m = jnp.sum(m * mask_c, axis=2, keepdims=True)
        m_new = m - col_m * row_mn + mask_r * row_mn
        inv_new = inv - col_m * row_in + mask_r * row_in
        return m_new, inv_new

    _, a_inv = jax.lax.fori_loop(0, _d, gj_step, (a, eye))

    for k in range(_K):
        u = us_ref[k]
        v_ref[k] = jax.lax.dot_general(u, a_inv[k], (((1,), (0,)), ((), ())),
                                       preferred_element_type=jnp.float32,
                                       precision=_PREC)


def _apply_kernel(x_ref, us_ref, v_ref, o_ref):
    t = jax.lax.dot_general(x_ref[...], us_ref[0], (((1,), (0,)), ((), ())),
                            preferred_element_type=jnp.float32,
                            precision=_PREC_APPLY)
    o_ref[0] = jax.lax.dot_general(t, v_ref[0], (((1,), (1,)), ((), ())),
                                   preferred_element_type=jnp.float32,
                                   precision=_PREC_APPLY)


def kernel(x, Us):
    batch = x.shape[0]
    n_bt = batch // _BT

    v = pl.pallas_call(
        _prep_kernel,
        out_shape=jax.ShapeDtypeStruct((_K, _D, _d), jnp.float32),
    )(Us)

    out = pl.pallas_call(
        _apply_kernel,
        grid=(n_bt, _K),
        in_specs=[
            pl.BlockSpec((_BT, _D), lambda i, k: (i, 0)),
            pl.BlockSpec((1, _D, _d), lambda i, k: (k, 0, 0)),
            pl.BlockSpec((1, _D, _d), lambda i, k: (k, 0, 0)),
        ],
        out_specs=pl.BlockSpec((1, _BT, _D), lambda i, k: (k, i, 0)),
        out_shape=jax.ShapeDtypeStruct((_K, batch, _D), jnp.float32),
    )(x, Us, v)
    return out


# speedup vs baseline: 4.6367x; 1.0187x over previous
import jax
import jax.numpy as jnp
from jax.experimental import pallas as pl

_K = 16
_D = 1024
_d = 64
_BT = 4096

_PREC = jax.lax.Precision.HIGHEST
_PREC_APPLY = jax.lax.Precision.DEFAULT


def _prep_kernel(us_ref, v_ref):
    a_list = []
    for k in range(_K):
        u = us_ref[k]
        a_list.append(
            jax.lax.dot_general(u, u, (((0,), (0,)), ((), ())),
                                preferred_element_type=jnp.float32,
                                precision=_PREC))
    a = jnp.stack(a_list, axis=0)

    rows = jax.lax.broadcasted_iota(jnp.int32, (_K, _d, _d), 1)
    cols = jax.lax.broadcasted_iota(jnp.int32, (_K, _d, _d), 2)
    eye = (rows == cols).astype(jnp.float32)

    def gj_step(j, carry):
        m, inv = carry
        mask_r = (rows == j).astype(jnp.float32)
        mask_c = (cols == j).astype(jnp.float32)
        row_m = jnp.sum(m * mask_r, axis=1, keepdims=True)
        row_i = jnp.sum(inv * mask_r, axis=1, keepdims=True)
        piv = jnp.sum(row_m * mask_c[:, :1, :], axis=2, keepdims=True)
        inv_piv = 1.0 / piv
        row_mn = row_m * inv_piv
        row_in = row_i * inv_piv
        col_m = jnp.sum(m * mask_c, axis=2, keepdims=True)
        m_new = m - col_m * row_mn + mask_r * row_mn
        inv_new = inv - col_m * row_in + mask_r * row_in
        return m_new, inv_new

    _, a_inv = jax.lax.fori_loop(0, _d, gj_step, (a, eye))

    for k in range(_K):
        u = us_ref[k]
        v_ref[k] = jax.lax.dot_general(u, a_inv[k], (((1,), (0,)), ((), ())),
                                       preferred_element_type=jnp.float32,
                                       precision=_PREC)


def _apply_kernel(x_ref, us_ref, v_ref, o_ref):
    t = jax.lax.dot_general(x_ref[...], us_ref[0], (((1,), (0,)), ((), ())),
                            preferred_element_type=jnp.float32,
                            precision=_PREC_APPLY)
    o_ref[0] = jax.lax.dot_general(t, v_ref[0], (((1,), (1,)), ((), ())),
                                   preferred_element_type=jnp.float32,
                                   precision=_PREC_APPLY)


def kernel(x, Us):
    batch = x.shape[0]
    n_bt = batch // _BT

    v = pl.pallas_call(
        _prep_kernel,
        out_shape=jax.ShapeDtypeStruct((_K, _D, _d), jnp.float32),
    )(Us)

    out = pl.pallas_call(
        _apply_kernel,
        grid=(n_bt, _K),
        in_specs=[
            pl.BlockSpec((_BT, _D), lambda i, k: (i, 0)),
            pl.BlockSpec((1, _D, _d), lambda i, k: (k, 0, 0)),
            pl.BlockSpec((1, _D, _d), lambda i, k: (k, 0, 0)),
        ],
        out_specs=pl.BlockSpec((1, _BT, _D), lambda i, k: (k, i, 0)),
        out_shape=jax.ShapeDtypeStruct((_K, batch, _D), jnp.float32),
    )(x, Us, v)
    return out


# speedup vs baseline: 6.3180x; 1.3626x over previous
import jax
import jax.numpy as jnp
from jax.experimental import pallas as pl

_K = 16
_D = 1024
_d = 64
_BT = 256

_PREC = jax.lax.Precision.HIGHEST
_PREC_APPLY = jax.lax.Precision.DEFAULT


def _prep_kernel(us_ref, v_ref):
    a_list = []
    for k in range(_K):
        u = us_ref[k]
        a_list.append(
            jax.lax.dot_general(u, u, (((0,), (0,)), ((), ())),
                                preferred_element_type=jnp.float32,
                                precision=_PREC))
    a = jnp.stack(a_list, axis=0)

    rows = jax.lax.broadcasted_iota(jnp.int32, (_K, _d, 2 * _d), 1)
    cols = jax.lax.broadcasted_iota(jnp.int32, (_K, _d, 2 * _d), 2)
    eye = (jax.lax.broadcasted_iota(jnp.int32, (_K, _d, _d), 1)
           == jax.lax.broadcasted_iota(jnp.int32, (_K, _d, _d), 2)
           ).astype(jnp.float32)
    aug0 = jnp.concatenate([a, eye], axis=2)

    def gj_step(j, aug):
        mask_r = (rows == j).astype(jnp.float32)
        row = jnp.sum(aug * mask_r, axis=1, keepdims=True)
        mask_cj = (cols == j).astype(jnp.float32)
        piv = jnp.sum(row * mask_cj[:, :1, :], axis=2, keepdims=True)
        row_n = row * (1.0 / piv)
        col = jnp.sum(aug * mask_cj, axis=2, keepdims=True)
        return aug - col * row_n + mask_r * row_n

    aug = jax.lax.fori_loop(0, _d, gj_step, aug0)
    a_inv = aug[:, :, _d:]

    for k in range(_K):
        u = us_ref[k]
        v_ref[k] = jax.lax.dot_general(u, a_inv[k], (((1,), (0,)), ((), ())),
                                       preferred_element_type=jnp.float32,
                                       precision=_PREC)


def _apply_kernel(x_ref, ucat_ref, v_ref, o_ref):
    t = jax.lax.dot_general(x_ref[...], ucat_ref[...], (((1,), (0,)), ((), ())),
                            preferred_element_type=jnp.float32,
                            precision=_PREC_APPLY)
    for k in range(_K):
        tk = jax.lax.slice(t, (0, k * _d), (t.shape[0], (k + 1) * _d))
        o_ref[k] = jax.lax.dot_general(tk, v_ref[k], (((1,), (1,)), ((), ())),
                                       preferred_element_type=jnp.float32,
                                       precision=_PREC_APPLY)


def kernel(x, Us):
    batch = x.shape[0]
    n_bt = batch // _BT

    v = pl.pallas_call(
        _prep_kernel,
        out_shape=jax.ShapeDtypeStruct((_K, _D, _d), jnp.float32),
    )(Us)

    u_cat = jnp.transpose(Us, (1, 0, 2)).reshape(_D, _K * _d)

    out = pl.pallas_call(
        _apply_kernel,
        grid=(n_bt,),
        in_specs=[
            pl.BlockSpec((_BT, _D), lambda i: (i, 0)),
            pl.BlockSpec((_D, _K * _d), lambda i: (0, 0)),
            pl.BlockSpec((_K, _D, _d), lambda i: (0, 0, 0)),
        ],
        out_specs=pl.BlockSpec((_K, _BT, _D), lambda i: (0, i, 0)),
        out_shape=jax.ShapeDtypeStruct((_K, batch, _D), jnp.float32),
    )(x, u_cat, v)
    return out


# speedup vs baseline: 6.5694x; 1.0398x over previous
import jax
import jax.numpy as jnp
from jax.experimental import pallas as pl
from jax.experimental.pallas import tpu as pltpu

_K = 16
_D = 1024
_d = 64
_BT = 256

_PREC = jax.lax.Precision.DEFAULT
_PREC_APPLY = jax.lax.Precision.DEFAULT


def _prep_kernel(us_ref, v_ref, aug_ref):
    a_list = []
    for k in range(_K):
        u = us_ref[k]
        a_list.append(
            jax.lax.dot_general(u, u, (((0,), (0,)), ((), ())),
                                preferred_element_type=jnp.float32,
                                precision=_PREC))
    a = jnp.stack(a_list, axis=0)

    aug_ref[...] = a
    rows_col = jax.lax.broadcasted_iota(jnp.int32, (_K, _d, 1), 1)
    cols_row = jax.lax.broadcasted_iota(jnp.int32, (_K, 1, _d), 2)

    def sweep_step(j, _):
        cur = aug_ref[...]
        row = aug_ref[:, pl.ds(j, 1), :]
        ej_row = (cols_row == j).astype(jnp.float32)
        ej_col = (rows_col == j).astype(jnp.float32)
        piv = jnp.sum(row * ej_row, axis=2, keepdims=True)
        row_adj = row - ej_row
        col_adj = jnp.transpose(row_adj, (0, 2, 1))
        aug_ref[...] = (cur - col_adj * (row_adj * (1.0 / piv))
                        - 2.0 * (ej_col * ej_row))
        return 0

    jax.lax.fori_loop(0, _d, sweep_step, 0)
    a_inv = -aug_ref[...]

    for k in range(_K):
        u = us_ref[k]
        v_ref[k] = jax.lax.dot_general(u, a_inv[k], (((1,), (0,)), ((), ())),
                                       preferred_element_type=jnp.float32,
                                       precision=_PREC)


def _apply_kernel(x_ref, ucat_ref, v_ref, o_ref):
    t = jax.lax.dot_general(x_ref[...], ucat_ref[...], (((1,), (0,)), ((), ())),
                            preferred_element_type=jnp.float32,
                            precision=_PREC_APPLY)
    for k in range(_K):
        tk = jax.lax.slice(t, (0, k * _d), (t.shape[0], (k + 1) * _d))
        o_ref[k] = jax.lax.dot_general(tk, v_ref[k], (((1,), (1,)), ((), ())),
                                       preferred_element_type=jnp.float32,
                                       precision=_PREC_APPLY)


def kernel(x, Us):
    batch = x.shape[0]
    n_bt = batch // _BT

    v = pl.pallas_call(
        _prep_kernel,
        out_shape=jax.ShapeDtypeStruct((_K, _D, _d), jnp.float32),
        scratch_shapes=[pltpu.VMEM((_K, _d, _d), jnp.float32)],
    )(Us)

    u_cat = jnp.transpose(Us, (1, 0, 2)).reshape(_D, _K * _d)

    out = pl.pallas_call(
        _apply_kernel,
        grid=(n_bt,),
        in_specs=[
            pl.BlockSpec((_BT, _D), lambda i: (i, 0)),
            pl.BlockSpec((_D, _K * _d), lambda i: (0, 0)),
            pl.BlockSpec((_K, _D, _d), lambda i: (0, 0, 0)),
        ],
        out_specs=pl.BlockSpec((_K, _BT, _D), lambda i: (0, i, 0)),
        out_shape=jax.ShapeDtypeStruct((_K, batch, _D), jnp.float32),
    )(x, u_cat, v)
    return out


# speedup vs baseline: 6.6991x; 1.0197x over previous
import jax
import jax.numpy as jnp
from jax.experimental import pallas as pl
from jax.experimental.pallas import tpu as pltpu

_K = 16
_D = 1024
_d = 64
_BT = 256

_PREC = jax.lax.Precision.DEFAULT
_PREC_APPLY = jax.lax.Precision.DEFAULT


def _prep_kernel(us_ref, v_ref, ucat_ref, aug_ref):
    a_list = []
    for k in range(_K):
        u = us_ref[k]
        a_list.append(
            jax.lax.dot_general(u, u, (((0,), (0,)), ((), ())),
                                preferred_element_type=jnp.float32,
                                precision=_PREC))
    a = jnp.stack(a_list, axis=0)

    aug_ref[...] = a
    cols_row = jax.lax.broadcasted_iota(jnp.int32, (_K, 1, _d), 2)
    eye = (jax.lax.broadcasted_iota(jnp.int32, (_K, _d, _d), 1)
           == jax.lax.broadcasted_iota(jnp.int32, (_K, _d, _d), 2)
           ).astype(jnp.float32)

    def sweep_step(j, _):
        cur = aug_ref[...]
        row = aug_ref[:, pl.ds(j, 1), :]
        ej_row = (cols_row == j).astype(jnp.float32)
        piv = jnp.sum(row * ej_row, axis=2, keepdims=True)
        row_adj = row - ej_row
        col_adj = jnp.transpose(row_adj, (0, 2, 1))
        aug_ref[...] = cur - col_adj * (row_adj * (1.0 / piv))
        return 0

    jax.lax.fori_loop(0, _d, sweep_step, 0)
    a_inv = 2.0 * eye - aug_ref[...]

    for k in range(_K):
        u = us_ref[k]
        v_ref[k] = jax.lax.dot_general(u, a_inv[k], (((1,), (0,)), ((), ())),
                                       preferred_element_type=jnp.float32,
                                       precision=_PREC)
        ucat_ref[:, k * _d:(k + 1) * _d] = u


def _apply_kernel(x_ref, ucat_ref, v_ref, o_ref):
    t = jax.lax.dot_general(x_ref[...], ucat_ref[...], (((1,), (0,)), ((), ())),
                            preferred_element_type=jnp.float32,
                            precision=_PREC_APPLY)
    for k in range(_K):
        tk = jax.lax.slice(t, (0, k * _d), (t.shape[0], (k + 1) * _d))
        o_ref[k] = jax.lax.dot_general(tk, v_ref[k], (((1,), (1,)), ((), ())),
                                       preferred_element_type=jnp.float32,
                                       precision=_PREC_APPLY)


def kernel(x, Us):
    batch = x.shape[0]
    n_bt = batch // _BT

    v, u_cat = pl.pallas_call(
        _prep_kernel,
        out_shape=(jax.ShapeDtypeStruct((_K, _D, _d), jnp.float32),
                   jax.ShapeDtypeStruct((_D, _K * _d), jnp.float32)),
        scratch_shapes=[pltpu.VMEM((_K, _d, _d), jnp.float32)],
    )(Us)

    out = pl.pallas_call(
        _apply_kernel,
        grid=(n_bt,),
        in_specs=[
            pl.BlockSpec((_BT, _D), lambda i: (i, 0)),
            pl.BlockSpec((_D, _K * _d), lambda i: (0, 0)),
            pl.BlockSpec((_K, _D, _d), lambda i: (0, 0, 0)),
        ],
        out_specs=pl.BlockSpec((_K, _BT, _D), lambda i: (0, i, 0)),
        out_shape=jax.ShapeDtypeStruct((_K, batch, _D), jnp.float32),
    )(x, u_cat, v)
    return out
